# baseline (speedup 1.0000x reference)
import jax
import jax.numpy as jnp
from jax.experimental import pallas as pl
from jax.experimental.pallas import tpu as pltpu

_D = 3


def _heads_body(x0, x1, x2, w0, b0, w1, b1, w2, b2, o0, o1, o2,
                ia0, ia1, ib1_, ib2_, oa0, oa1, ob1_, ob2_, isem, osem):
    B = x0.shape[0]
    h0 = x0.shape[1] // 2
    dn = (((1,), (1,)), ((), ()))

    ibufs = (ia0, ia1, ib1_, ib2_)
    obufs = (oa0, oa1, ob1_, ob2_)

    def in_copy(st, b):
        slot = b % _D
        if st == 0:
            src = x0.at[b, :h0]
        elif st == 1:
            src = x0.at[b, h0:]
        elif st == 2:
            src = x1.at[b]
        else:
            src = x2.at[b]
        return pltpu.make_async_copy(src, ibufs[st].at[slot],
                                     isem.at[st, slot])

    def out_copy(st, b):
        slot = b % _D
        if st == 0:
            dst = o0.at[b, :h0]
        elif st == 1:
            dst = o0.at[b, h0:]
        elif st == 2:
            dst = o1.at[b]
        else:
            dst = o2.at[b]
        return pltpu.make_async_copy(obufs[st].at[slot], dst,
                                     osem.at[st, slot])

    wb = ((w0, b0), (w0, b0), (w1, b1), (w2, b2))

    for b in range(_D):
        for st in range(4):
            in_copy(st, b).start()

    for b in range(B):
        slot = b % _D
        for st in range(4):
            in_copy(st, b).wait()
            if b >= _D:
                out_copy(st, b - _D).wait()
            w, bia = wb[st]
            obufs[st][slot] = jax.lax.dot_general(
                ibufs[st][slot], w[...], dn,
                preferred_element_type=jnp.float32) + bia[...]
            out_copy(st, b).start()
            if b + _D < B:
                in_copy(st, b + _D).start()

    for b in range(max(0, B - _D), B):
        for st in range(4):
            out_copy(st, b).wait()


def kernel(feats_0, feats_1, feats_2, W0, b0, W1, b1, W2, b2):
    B = feats_0.shape[0]
    shapes = [feats_0.shape, feats_1.shape, feats_2.shape]
    xs = [jnp.transpose(f, (0, 2, 3, 1)).reshape(
              f.shape[0], f.shape[2] * f.shape[3], f.shape[1])
          for f in (feats_0, feats_1, feats_2)]
    ws = [W0, W1, W2]
    bs = [b.reshape(1, -1) for b in (b0, b1, b2)]
    out_dim = W0.shape[0]

    hbm = pl.BlockSpec(memory_space=pltpu.MemorySpace.HBM)
    vmem = pl.BlockSpec(memory_space=pltpu.MemorySpace.VMEM)

    out_shapes = [jax.ShapeDtypeStruct((B, x.shape[1], out_dim), jnp.float32)
                  for x in xs]

    h0 = xs[0].shape[1] // 2
    scratch = [
        pltpu.VMEM((_D, h0, xs[0].shape[2]), jnp.float32),
        pltpu.VMEM((_D, h0, xs[0].shape[2]), jnp.float32),
        pltpu.VMEM((_D, xs[1].shape[1], xs[1].shape[2]), jnp.float32),
        pltpu.VMEM((_D, xs[2].shape[1], xs[2].shape[2]), jnp.float32),
        pltpu.VMEM((_D, h0, out_dim), jnp.float32),
        pltpu.VMEM((_D, h0, out_dim), jnp.float32),
        pltpu.VMEM((_D, xs[1].shape[1], out_dim), jnp.float32),
        pltpu.VMEM((_D, xs[2].shape[1], out_dim), jnp.float32),
        pltpu.SemaphoreType.DMA((4, _D)),
        pltpu.SemaphoreType.DMA((4, _D)),
    ]

    outs = pl.pallas_call(
        _heads_body,
        in_specs=[hbm, hbm, hbm] + [vmem] * 6,
        out_specs=[hbm, hbm, hbm],
        out_shape=out_shapes,
        scratch_shapes=scratch,
        compiler_params=pltpu.CompilerParams(
            vmem_limit_bytes=60 * 1024 * 1024),
    )(xs[0], xs[1], xs[2], ws[0], bs[0], ws[1], bs[1], ws[2], bs[2])

    return tuple(
        jnp.transpose(o.reshape(s[0], s[2], s[3], out_dim), (0, 3, 1, 2))
        for o, s in zip(outs, shapes)
    )
